# baseline (speedup 1.0000x reference)
import functools
import math

import jax
import jax.numpy as jnp
from jax import lax
from jax.experimental import pallas as pl
from jax.experimental.pallas import tpu as pltpu
from jax.experimental.pallas import tpu_sc as plsc

D_MODEL = 128
SCALE = math.sqrt(float(D_MODEL))

NUM_CORES = 2
NUM_SUBCORES = 16
NW = NUM_CORES * NUM_SUBCORES

CHUNK = 128
NG = 4
NO = 2


def _make_gather(vocab: int, batch: int):
    assert batch % (NW * CHUNK * NG) == 0
    rows_per_w = batch // NW
    n_chunks = rows_per_w // CHUNK
    n_rings = n_chunks // NG

    mesh = plsc.VectorSubcoreMesh(
        core_axis_name="c", subcore_axis_name="s",
        num_cores=NUM_CORES, num_subcores=NUM_SUBCORES,
    )

    @functools.partial(
        pl.kernel,
        out_type=jax.ShapeDtypeStruct((batch, D_MODEL), jnp.float32),
        mesh=mesh,
        scratch_types=[
            pltpu.VMEM((n_chunks, CHUNK), jnp.int32),
            [pltpu.VMEM((CHUNK, D_MODEL), jnp.float32) for _ in range(NG)],
            [pltpu.VMEM((CHUNK, D_MODEL), jnp.float32) for _ in range(NO)],
            [pltpu.SemaphoreType.DMA for _ in range(NG)],
            [pltpu.SemaphoreType.DMA for _ in range(NO)],
        ],
    )
    def gather_kernel(table_hbm, idx_hbm, out_hbm, idx_v, gbufs, wbufs,
                      gsems, osems):
        wid = lax.axis_index("s") * NUM_CORES + lax.axis_index("c")
        out_row0 = wid * rows_per_w

        pltpu.sync_copy(idx_hbm.at[pl.ds(wid * n_chunks, n_chunks)], idx_v)

        def gather_chunk(g, b):
            return pltpu.async_copy(
                table_hbm.at[idx_v.at[g]], gbufs[b], gsems[b])

        def write_desc(g, b):
            return pltpu.make_async_copy(
                wbufs[b], out_hbm.at[pl.ds(out_row0 + g * CHUNK, CHUNK)],
                osems[b])

        for b in range(NG):
            gather_chunk(b, b)

        @pl.loop(0, n_rings)
        def _ring(it):
            for b in range(NG):
                g = it * NG + b
                bo = b % NO
                pltpu.make_async_copy(
                    table_hbm.at[idx_v.at[g]], gbufs[b], gsems[b]).wait()

                if b < NO:
                    @pl.when(it > 0)
                    def _():
                        write_desc(g - NO, bo).wait()
                else:
                    write_desc(g - NO, bo).wait()

                @plsc.parallel_loop(0, CHUNK, unroll=4)
                def _scale(i):
                    for c in range(D_MODEL // 16):
                        wbufs[bo][i, pl.ds(c * 16, 16)] = (
                            gbufs[b][i, pl.ds(c * 16, 16)] * SCALE
                        )

                write_desc(g, bo).start()

                @pl.when(it < n_rings - 1)
                def _():
                    gather_chunk(g + NG, b)

        for b in range(NO):
            write_desc(n_chunks - NO + b, (n_chunks - NO + b) % NO).wait()

    return gather_kernel


def kernel(tokens, embedding):
    b, h = tokens.shape
    batch = b * h
    idx2d = tokens.reshape(batch // CHUNK, CHUNK).astype(jnp.int32)
    out = _make_gather(embedding.shape[0], batch)(embedding, idx2d)
    return out.reshape(b, h, D_MODEL)
